# baseline (device time: 9402 ns/iter reference)
import jax
import jax.numpy as jnp
from jax import lax
from jax.experimental import pallas as pl
from jax.experimental.pallas import tpu as pltpu

N_CHUNKS = 4


def kernel(x):
    m_per, n = x.shape
    rows = m_per // N_CHUNKS

    def body(x_ref, out_ref, xq_ref, rq_ref, xs_ref, rs_ref, deq_ref,
             send_sem, recv_sem, ssend_sem, srecv_sem, copy_sem,
             wb_sem):
        my_x = lax.axis_index("x")
        my_y = lax.axis_index("y")
        my_z = lax.axis_index("z")
        nbr = (my_x, 1 - my_y, my_z)
        mine = my_y * m_per
        theirs = (1 - my_y) * m_per

        def quantize(c):
            xc = x_ref[pl.ds(c * rows, rows), :]
            s = jnp.max(jnp.abs(xc)) / 127.0
            xq_ref[pl.ds(c * rows, rows), :] = jnp.clip(
                jnp.rint(xc / s), -127.0, 127.0
            ).astype(jnp.int8)
            xs_ref[pl.ds(c * 8, 8), :] = jnp.full((8, 128), s, jnp.float32)

        quantize(0)
        barrier_sem = pltpu.get_barrier_semaphore()
        pl.semaphore_signal(
            barrier_sem, inc=1, device_id=nbr,
            device_id_type=pl.DeviceIdType.MESH,
        )
        pl.semaphore_wait(barrier_sem, 1)

        rdmas = []
        for c in range(N_CHUNKS):
            rdma = pltpu.make_async_remote_copy(
                src_ref=xq_ref.at[pl.ds(c * rows, rows), :],
                dst_ref=rq_ref.at[pl.ds(c * rows, rows), :],
                send_sem=send_sem.at[c],
                recv_sem=recv_sem.at[c],
                device_id=nbr,
                device_id_type=pl.DeviceIdType.MESH,
            )
            rdma.start()
            srdma = pltpu.make_async_remote_copy(
                src_ref=xs_ref.at[pl.ds(c * 8, 8), :],
                dst_ref=rs_ref.at[pl.ds(c * 8, 8), :],
                send_sem=ssend_sem.at[c],
                recv_sem=srecv_sem.at[c],
                device_id=nbr,
                device_id_type=pl.DeviceIdType.MESH,
            )
            srdma.start()
            rdmas.append((rdma, srdma))
            if c + 1 < N_CHUNKS:
                quantize(c + 1)

        local = pltpu.make_async_copy(
            x_ref,
            out_ref.at[pl.ds(mine, m_per), :],
            copy_sem,
        )
        local.start()

        wbs = []
        for c in range(N_CHUNKS):
            rdma, srdma = rdmas[c]
            rdma.wait()
            srdma.wait()
            s_recv = rs_ref[pl.ds(c * 8, 8), :][0:1, 0:1]
            deq_ref[pl.ds(c * rows, rows), :] = (
                rq_ref[pl.ds(c * rows, rows), :].astype(jnp.float32) * s_recv
            )
            wb = pltpu.make_async_copy(
                deq_ref.at[pl.ds(c * rows, rows), :],
                out_ref.at[pl.ds(theirs + c * rows, rows), :],
                wb_sem.at[c],
            )
            wb.start()
            wbs.append(wb)
        for wb in wbs:
            wb.wait()
        local.wait()

    return pl.pallas_call(
        body,
        out_shape=jax.ShapeDtypeStruct((2 * m_per, n), x.dtype),
        in_specs=[pl.BlockSpec(memory_space=pltpu.VMEM)],
        out_specs=pl.BlockSpec(memory_space=pl.ANY),
        scratch_shapes=[
            pltpu.VMEM((m_per, n), jnp.int8),
            pltpu.VMEM((m_per, n), jnp.int8),
            pltpu.VMEM((8 * N_CHUNKS, 128), jnp.float32),
            pltpu.VMEM((8 * N_CHUNKS, 128), jnp.float32),
            pltpu.VMEM((m_per, n), jnp.float32),
            pltpu.SemaphoreType.DMA((N_CHUNKS,)),
            pltpu.SemaphoreType.DMA((N_CHUNKS,)),
            pltpu.SemaphoreType.DMA((N_CHUNKS,)),
            pltpu.SemaphoreType.DMA((N_CHUNKS,)),
            pltpu.SemaphoreType.DMA,
            pltpu.SemaphoreType.DMA((N_CHUNKS,)),
        ],
        compiler_params=pltpu.CompilerParams(collective_id=0),
    )(x)


# device time: 9259 ns/iter; 1.0154x vs baseline; 1.0154x over previous
import jax
import jax.numpy as jnp
from jax import lax
from jax.experimental import pallas as pl
from jax.experimental.pallas import tpu as pltpu

N_CHUNKS = 2


def kernel(x):
    m_per, n = x.shape
    rows = m_per // N_CHUNKS

    def body(x_ref, out_ref, xq_ref, rq_ref, xs_ref, rs_ref,
             send_sem, recv_sem, ssend_sem, srecv_sem):
        my_x = lax.axis_index("x")
        my_y = lax.axis_index("y")
        my_z = lax.axis_index("z")
        nbr = (my_x, 1 - my_y, my_z)
        mine = my_y * m_per
        theirs = (1 - my_y) * m_per

        def quantize(c):
            xc = x_ref[pl.ds(c * rows, rows), :]
            s = jnp.max(jnp.abs(xc)) / 127.0
            xq_ref[pl.ds(c * rows, rows), :] = jnp.clip(
                jnp.rint(xc / s), -127.0, 127.0
            ).astype(jnp.int8)
            xs_ref[pl.ds(c * 8, 8), :] = jnp.full((8, 128), s, jnp.float32)

        quantize(0)
        barrier_sem = pltpu.get_barrier_semaphore()
        pl.semaphore_signal(
            barrier_sem, inc=1, device_id=nbr,
            device_id_type=pl.DeviceIdType.MESH,
        )
        pl.semaphore_wait(barrier_sem, 1)

        rdmas = []
        for c in range(N_CHUNKS):
            rdma = pltpu.make_async_remote_copy(
                src_ref=xq_ref.at[pl.ds(c * rows, rows), :],
                dst_ref=rq_ref.at[pl.ds(c * rows, rows), :],
                send_sem=send_sem.at[c],
                recv_sem=recv_sem.at[c],
                device_id=nbr,
                device_id_type=pl.DeviceIdType.MESH,
            )
            rdma.start()
            srdma = pltpu.make_async_remote_copy(
                src_ref=xs_ref.at[pl.ds(c * 8, 8), :],
                dst_ref=rs_ref.at[pl.ds(c * 8, 8), :],
                send_sem=ssend_sem.at[c],
                recv_sem=srecv_sem.at[c],
                device_id=nbr,
                device_id_type=pl.DeviceIdType.MESH,
            )
            srdma.start()
            rdmas.append((rdma, srdma))
            if c + 1 < N_CHUNKS:
                quantize(c + 1)

        out_ref[pl.ds(mine, m_per), :] = x_ref[...]

        for c in range(N_CHUNKS):
            rdma, srdma = rdmas[c]
            rdma.wait()
            srdma.wait()
            s_recv = rs_ref[pl.ds(c * 8, 8), :][0:1, 0:1]
            out_ref[pl.ds(theirs + c * rows, rows), :] = (
                rq_ref[pl.ds(c * rows, rows), :].astype(jnp.float32) * s_recv
            )

    return pl.pallas_call(
        body,
        out_shape=jax.ShapeDtypeStruct((2 * m_per, n), x.dtype),
        in_specs=[pl.BlockSpec(memory_space=pltpu.VMEM)],
        out_specs=pl.BlockSpec(memory_space=pltpu.VMEM),
        scratch_shapes=[
            pltpu.VMEM((m_per, n), jnp.int8),
            pltpu.VMEM((m_per, n), jnp.int8),
            pltpu.VMEM((8 * N_CHUNKS, 128), jnp.float32),
            pltpu.VMEM((8 * N_CHUNKS, 128), jnp.float32),
            pltpu.SemaphoreType.DMA((N_CHUNKS,)),
            pltpu.SemaphoreType.DMA((N_CHUNKS,)),
            pltpu.SemaphoreType.DMA((N_CHUNKS,)),
            pltpu.SemaphoreType.DMA((N_CHUNKS,)),
        ],
        compiler_params=pltpu.CompilerParams(collective_id=0),
    )(x)
